# Initial kernel scaffold; baseline (speedup 1.0000x reference)
import jax
import jax.numpy as jnp
from jax.experimental import pallas as pl


def kernel(features, tables, W1, b1, W2, b2):
    raise NotImplementedError("write your pallas kernel here")


# speedup vs baseline: 8.0669x; 8.0669x over previous
import functools

import jax
import jax.numpy as jnp
from jax import lax
from jax.experimental import pallas as pl
from jax.experimental.pallas import tpu as pltpu
from jax.experimental.pallas import tpu_sc as plsc

F = 26
V = 100000
D = 32
B = 16384
H = 512
O = 128
IN = F * D

NW = 32
PER_W = B * F // NW
CH = 128
NCH = PER_W // CH
K = 8
NG = NCH // K

_sc_mesh = plsc.VectorSubcoreMesh(core_axis_name="c", subcore_axis_name="s")


@functools.partial(
    pl.kernel,
    mesh=_sc_mesh,
    out_type=jax.ShapeDtypeStruct((NW, NCH, CH, D), jnp.float32),
    scratch_types=[
        pltpu.VMEM((NCH, CH), jnp.int32),
        pltpu.VMEM((K, CH, D), jnp.float32),
        pltpu.SemaphoreType.DMA,
        pltpu.SemaphoreType.DMA,
    ],
    compiler_params=pltpu.CompilerParams(use_tc_tiling_on_sc=False),
)
def _sc_gather(tab_hbm, idx_hbm, out_hbm, idx_v, rows_v, gsem, ssem):
    wid = lax.axis_index("s") * 2 + lax.axis_index("c")
    pltpu.sync_copy(idx_hbm.at[wid], idx_v)

    @pl.loop(0, NG)
    def _group(g):
        gds = []
        for b in range(K):
            j = g * K + b
            gds.append(pltpu.async_copy(tab_hbm.at[idx_v.at[j]], rows_v.at[b], gsem))
        for d in gds:
            d.wait()
        sds = []
        for b in range(K):
            j = g * K + b
            sds.append(pltpu.async_copy(rows_v.at[b], out_hbm.at[wid, j], ssem))
        for d in sds:
            d.wait()


TB = 1024


def _mlp_body(x_ref, w1_ref, b1_ref, w2_ref, b2_ref, o_ref):
    h = jnp.dot(x_ref[...], w1_ref[...], preferred_element_type=jnp.float32)
    h = jnp.maximum(h + b1_ref[...], 0.0)
    o_ref[...] = jnp.dot(h, w2_ref[...], preferred_element_type=jnp.float32) + b2_ref[...]


_mlp = pl.pallas_call(
    _mlp_body,
    grid=(B // TB,),
    in_specs=[
        pl.BlockSpec((TB, IN), lambda i: (i, 0)),
        pl.BlockSpec((IN, H), lambda i: (0, 0)),
        pl.BlockSpec((1, H), lambda i: (0, 0)),
        pl.BlockSpec((H, O), lambda i: (0, 0)),
        pl.BlockSpec((1, O), lambda i: (0, 0)),
    ],
    out_specs=pl.BlockSpec((TB, O), lambda i: (i, 0)),
    out_shape=jax.ShapeDtypeStruct((B, O), jnp.float32),
)


def kernel(features, tables, W1, b1, W2, b2):
    offs = (jnp.arange(F, dtype=jnp.int32) * V)[None, :]
    flat_idx = (features.astype(jnp.int32) + offs).reshape(NW, NCH, CH)
    tab = tables.reshape(F * V, D)
    x = _sc_gather(tab, flat_idx)
    x = x.reshape(B, IN)
    return _mlp(x, W1, b1.reshape(1, H), W2, b2.reshape(1, O))


# speedup vs baseline: 8.2227x; 1.0193x over previous
import functools

import jax
import jax.numpy as jnp
from jax import lax
from jax.experimental import pallas as pl
from jax.experimental.pallas import tpu as pltpu
from jax.experimental.pallas import tpu_sc as plsc

F = 26
V = 100000
D = 32
B = 16384
H = 512
O = 128
F28 = 28
T = F28 // 4
NG = B // 8

NW = 32
G_PER_W = NG // NW
CH = 112
NCH = 2 * G_PER_W
NBUF = 2

_sc_mesh = plsc.VectorSubcoreMesh(core_axis_name="c", subcore_axis_name="s")


@functools.partial(
    pl.kernel,
    mesh=_sc_mesh,
    out_type=jax.ShapeDtypeStruct((NG, 56, 128), jnp.float32),
    scratch_types=[
        pltpu.VMEM((NCH, 128), jnp.int32),
        pltpu.VMEM((NBUF, 224, D), jnp.float32),
        [pltpu.SemaphoreType.DMA] * NBUF,
        [pltpu.SemaphoreType.DMA] * NBUF,
    ],
    compiler_params=pltpu.CompilerParams(use_tc_tiling_on_sc=False),
)
def _sc_gather(tab_hbm, idx_hbm, out_hbm, idx_v, rows_v, gsems, ssems):
    wid = lax.axis_index("s") * 2 + lax.axis_index("c")
    pltpu.sync_copy(idx_hbm.at[wid], idx_v)
    g_base = wid * G_PER_W

    def fire_gather(u, b):
        for h in range(2):
            pltpu.async_copy(
                tab_hbm.at[idx_v.at[2 * u + h, pl.ds(0, CH)]],
                rows_v.at[b, pl.ds(h * CH, CH)],
                gsems[b],
            )

    def drain_gather(b):
        for h in range(2):
            pltpu.make_async_copy(
                tab_hbm.at[idx_v.at[h, pl.ds(0, CH)]],
                rows_v.at[b, pl.ds(h * CH, CH)],
                gsems[b],
            ).wait()

    def fire_store(u, b):
        for j in range(4):
            pltpu.async_copy(
                rows_v.at[b, pl.ds(j * 56, 56)],
                out_hbm.at[g_base + u, slice(None), pl.ds(32 * j, 32)],
                ssems[b],
            )

    def drain_store(b):
        for j in range(4):
            pltpu.make_async_copy(
                rows_v.at[b, pl.ds(j * 56, 56)],
                out_hbm.at[0, slice(None), pl.ds(32 * j, 32)],
                ssems[b],
            ).wait()

    for b in range(NBUF - 1):
        fire_gather(b, b)

    @pl.loop(0, G_PER_W // NBUF)
    def _q(q):
        for b in range(NBUF):
            u = q * NBUF + b
            v = u + (NBUF - 1)
            bv = (b + NBUF - 1) % NBUF

            @pl.when(v < G_PER_W)
            def _():
                @pl.when(v >= NBUF)
                def _():
                    drain_store(bv)

                fire_gather(v, bv)

            drain_gather(b)
            fire_store(u, b)

    for b in range(NBUF):
        drain_store(b)


TB = 1024
GT = TB // 8


def _mlp_body(x_ref, w1_ref, b1_ref, w2_ref, b2_ref, o_ref):
    def xt(t):
        return x_ref[:, 8 * t:8 * (t + 1), :].reshape(TB, 128)

    acc = jnp.dot(xt(0), w1_ref[0], preferred_element_type=jnp.float32)
    for t in range(1, T):
        acc += jnp.dot(xt(t), w1_ref[t], preferred_element_type=jnp.float32)
    h = jnp.maximum(acc + b1_ref[...], 0.0)
    o_ref[...] = jnp.dot(h, w2_ref[...], preferred_element_type=jnp.float32) + b2_ref[...]


_mlp = pl.pallas_call(
    _mlp_body,
    grid=(B // TB,),
    in_specs=[
        pl.BlockSpec((GT, 56, 128), lambda i: (i, 0, 0)),
        pl.BlockSpec((T, 128, H), lambda i: (0, 0, 0)),
        pl.BlockSpec((1, H), lambda i: (0, 0)),
        pl.BlockSpec((H, O), lambda i: (0, 0)),
        pl.BlockSpec((1, O), lambda i: (0, 0)),
    ],
    out_specs=pl.BlockSpec((TB, O), lambda i: (i, 0)),
    out_shape=jax.ShapeDtypeStruct((B, O), jnp.float32),
)


def kernel(features, tables, W1, b1, W2, b2):
    offs = (jnp.arange(F, dtype=jnp.int32) * V)[None, :]
    fi = features.astype(jnp.int32) + offs
    fi28 = jnp.concatenate([fi, fi[:, 24:26]], axis=1)
    idx = fi28.reshape(NG, 8, T, 4).transpose(0, 3, 2, 1)
    idx = idx.reshape(NW, NCH, CH)
    idx = jnp.pad(idx, ((0, 0), (0, 0), (0, 128 - CH)))
    tab = tables.reshape(F * V, D)
    x3 = _sc_gather(tab, idx)
    w1p = jnp.concatenate(
        [W1, jnp.zeros((T * 128 - F * D, H), jnp.float32)]
    ).reshape(T, 128, H)
    return _mlp(x3, w1p, b1.reshape(1, H), W2, b2.reshape(1, O))


# speedup vs baseline: 11.7854x; 1.4333x over previous
import functools

import jax
import jax.numpy as jnp
from jax import lax
from jax.experimental import pallas as pl
from jax.experimental.pallas import tpu as pltpu
from jax.experimental.pallas import tpu_sc as plsc

F = 26
V = 100000
D = 32
B = 16384
H = 512
O = 128
F28 = 28
T = F28 // 4
NG = B // 8

NW = 32
G_PER_W = NG // NW
CH = 112
NCH = 2 * G_PER_W
NBUF = 2

_sc_mesh = plsc.VectorSubcoreMesh(core_axis_name="c", subcore_axis_name="s")


@functools.partial(
    pl.kernel,
    mesh=_sc_mesh,
    out_type=jax.ShapeDtypeStruct((NG, 56, 128), jnp.float32),
    scratch_types=[
        pltpu.VMEM((NCH, 128), jnp.int32),
        pltpu.VMEM((NBUF, 224, D), jnp.float32),
        [pltpu.SemaphoreType.DMA] * NBUF,
        [pltpu.SemaphoreType.DMA] * NBUF,
    ],
    compiler_params=pltpu.CompilerParams(use_tc_tiling_on_sc=False),
)
def _sc_gather(tab_hbm, idx_hbm, out_hbm, idx_v, rows_v, gsems, ssems):
    wid = lax.axis_index("s") * 2 + lax.axis_index("c")
    pltpu.sync_copy(idx_hbm.at[wid], idx_v)
    g_base = wid * G_PER_W

    def fire_gather(u, b):
        for h in range(2):
            pltpu.async_copy(
                tab_hbm.at[idx_v.at[2 * u + h, pl.ds(0, CH)]],
                rows_v.at[b, pl.ds(h * CH, CH)],
                gsems[b],
            )

    def drain_gather(b):
        for h in range(2):
            pltpu.make_async_copy(
                tab_hbm.at[idx_v.at[h, pl.ds(0, CH)]],
                rows_v.at[b, pl.ds(h * CH, CH)],
                gsems[b],
            ).wait()

    def fire_store(u, b):
        for j in range(4):
            pltpu.async_copy(
                rows_v.at[b, pl.ds(j * 56, 56)],
                out_hbm.at[g_base + u, slice(None), pl.ds(32 * j, 32)],
                ssems[b],
            )

    def drain_store(b):
        for j in range(4):
            pltpu.make_async_copy(
                rows_v.at[b, pl.ds(j * 56, 56)],
                out_hbm.at[0, slice(None), pl.ds(32 * j, 32)],
                ssems[b],
            ).wait()

    for b in range(NBUF - 1):
        fire_gather(b, b)

    @pl.loop(0, G_PER_W // NBUF)
    def _q(q):
        for b in range(NBUF):
            u = q * NBUF + b
            v = u + (NBUF - 1)
            bv = (b + NBUF - 1) % NBUF

            @pl.when(v < G_PER_W)
            def _():
                @pl.when(v >= NBUF)
                def _():
                    drain_store(bv)

                fire_gather(v, bv)

            drain_gather(b)
            fire_store(u, b)

    for b in range(NBUF):
        drain_store(b)


Q = V // 4


def _tr_body(in_ref, out_ref):
    x = in_ref[0]
    sc = Q // 8
    for c in range(8):
        out_ref[c * sc:(c + 1) * sc, :] = jnp.concatenate(
            [x[:, q * Q + c * sc:q * Q + (c + 1) * sc].T for q in range(4)],
            axis=1,
        )


_tr = pl.pallas_call(
    _tr_body,
    grid=(F,),
    in_specs=[pl.BlockSpec((1, D, V), lambda f: (f, 0, 0))],
    out_specs=pl.BlockSpec((Q, 4 * D), lambda f: (f, 0)),
    out_shape=jax.ShapeDtypeStruct((F * Q, 4 * D), jnp.float32),
    compiler_params=pltpu.CompilerParams(vmem_limit_bytes=100 * 1024 * 1024),
)


TB = 1024
GT = TB // 8


def _mlp_body(x_ref, w1_ref, b1_ref, w2_ref, b2_ref, o_ref):
    def xt(t):
        return x_ref[:, 8 * t:8 * (t + 1), :].reshape(TB, 128)

    acc = jnp.dot(xt(0), w1_ref[0], preferred_element_type=jnp.float32)
    for t in range(1, T):
        acc += jnp.dot(xt(t), w1_ref[t], preferred_element_type=jnp.float32)
    h = jnp.maximum(acc + b1_ref[...], 0.0)
    o_ref[...] = jnp.dot(h, w2_ref[...], preferred_element_type=jnp.float32) + b2_ref[...]


_mlp = pl.pallas_call(
    _mlp_body,
    grid=(B // TB,),
    in_specs=[
        pl.BlockSpec((GT, 56, 128), lambda i: (i, 0, 0)),
        pl.BlockSpec((T, 128, H), lambda i: (0, 0, 0)),
        pl.BlockSpec((1, H), lambda i: (0, 0)),
        pl.BlockSpec((H, O), lambda i: (0, 0)),
        pl.BlockSpec((1, O), lambda i: (0, 0)),
    ],
    out_specs=pl.BlockSpec((TB, O), lambda i: (i, 0)),
    out_shape=jax.ShapeDtypeStruct((B, O), jnp.float32),
)


def kernel(features, tables, W1, b1, W2, b2):
    v = features.astype(jnp.int32)
    offs = (jnp.arange(F, dtype=jnp.int32) * (4 * Q))[None, :]
    fi = offs + 4 * (v % Q) + v // Q
    fi28 = jnp.concatenate([fi, fi[:, 24:26]], axis=1)
    idx = fi28.reshape(NG, 8, T, 4).transpose(0, 3, 2, 1)
    idx = idx.reshape(NW, NCH, CH)
    idx = jnp.pad(idx, ((0, 0), (0, 0), (0, 128 - CH)))
    tab128 = _tr(jnp.transpose(tables, (0, 2, 1)))
    tab = tab128.reshape(F * V, D)
    x3 = _sc_gather(tab, idx)
    w1p = jnp.concatenate(
        [W1, jnp.zeros((T * 128 - F * D, H), jnp.float32)]
    ).reshape(T, 128, H)
    return _mlp(x3, w1p, b1.reshape(1, H), W2, b2.reshape(1, O))


# speedup vs baseline: 23.0048x; 1.9520x over previous
import functools

import jax
import jax.numpy as jnp
from jax import lax
from jax.experimental import pallas as pl
from jax.experimental.pallas import tpu as pltpu
from jax.experimental.pallas import tpu_sc as plsc

F = 26
V = 100000
D = 32
B = 16384
H = 512
O = 128
F28 = 28
T = F28 // 4
NG = B // 8

NW = 32
G_PER_W = NG // NW
CH = 112
NCH = 2 * G_PER_W
NBUF = 2

_sc_mesh = plsc.VectorSubcoreMesh(core_axis_name="c", subcore_axis_name="s")


@functools.partial(
    pl.kernel,
    mesh=_sc_mesh,
    out_type=jax.ShapeDtypeStruct((NG, 56, 128), jnp.float32),
    scratch_types=[
        pltpu.VMEM((NCH, 128), jnp.int32),
        pltpu.VMEM((NBUF, 224, D), jnp.float32),
        [pltpu.SemaphoreType.DMA] * NBUF,
        [pltpu.SemaphoreType.DMA] * NBUF,
    ],
    compiler_params=pltpu.CompilerParams(use_tc_tiling_on_sc=False),
)
def _sc_gather(tab_hbm, idx_hbm, out_hbm, idx_v, rows_v, gsems, ssems):
    wid = lax.axis_index("s") * 2 + lax.axis_index("c")
    pltpu.sync_copy(idx_hbm.at[wid], idx_v)
    g_base = wid * G_PER_W

    def fire_gather(u, b):
        for h in range(2):
            pltpu.async_copy(
                tab_hbm.at[idx_v.at[2 * u + h, pl.ds(0, CH)]],
                rows_v.at[b, pl.ds(h * CH, CH)],
                gsems[b],
            )

    def drain_gather(b):
        for h in range(2):
            pltpu.make_async_copy(
                tab_hbm.at[idx_v.at[h, pl.ds(0, CH)]],
                rows_v.at[b, pl.ds(h * CH, CH)],
                gsems[b],
            ).wait()

    def fire_store(u, b):
        for j in range(4):
            pltpu.async_copy(
                rows_v.at[b, pl.ds(j * 56, 56)],
                out_hbm.at[g_base + u, slice(None), pl.ds(32 * j, 32)],
                ssems[b],
            )

    def drain_store(b):
        for j in range(4):
            pltpu.make_async_copy(
                rows_v.at[b, pl.ds(j * 56, 56)],
                out_hbm.at[0, slice(None), pl.ds(32 * j, 32)],
                ssems[b],
            ).wait()

    for b in range(NBUF - 1):
        fire_gather(b, b)

    @pl.loop(0, G_PER_W // NBUF)
    def _q(q):
        for b in range(NBUF):
            u = q * NBUF + b
            v = u + (NBUF - 1)
            bv = (b + NBUF - 1) % NBUF

            @pl.when(v < G_PER_W)
            def _():
                @pl.when(v >= NBUF)
                def _():
                    drain_store(bv)

                fire_gather(v, bv)

            drain_gather(b)
            fire_store(u, b)

    for b in range(NBUF):
        drain_store(b)


QP = 25088
KW = 3584


def _tr_body(a_ref, b_ref, c_ref, d_ref, out_ref):
    z = jnp.concatenate(
        [a_ref[0], b_ref[0], c_ref[0], d_ref[0]], axis=0
    )
    out_ref[...] = z.T


_tr = pl.pallas_call(
    _tr_body,
    grid=(F, QP // KW),
    in_specs=[
        pl.BlockSpec((1, D, KW), (lambda f, k, q=q: (f, 0, (QP // KW) * q + k)))
        for q in range(4)
    ],
    out_specs=pl.BlockSpec((KW, 128), lambda f, k: ((QP // KW) * f + k, 0)),
    out_shape=jax.ShapeDtypeStruct((F * QP, 128), jnp.float32),
)


TB = 1024
GT = TB // 8


def _mlp_body(x_ref, w1_ref, b1_ref, w2_ref, b2_ref, o_ref):
    def xt(t):
        return x_ref[:, 8 * t:8 * (t + 1), :].reshape(TB, 128)

    acc = jnp.dot(xt(0), w1_ref[0], preferred_element_type=jnp.float32)
    for t in range(1, T):
        acc += jnp.dot(xt(t), w1_ref[t], preferred_element_type=jnp.float32)
    h = jnp.maximum(acc + b1_ref[...], 0.0)
    o_ref[...] = jnp.dot(h, w2_ref[...], preferred_element_type=jnp.float32) + b2_ref[...]


_mlp = pl.pallas_call(
    _mlp_body,
    grid=(B // TB,),
    in_specs=[
        pl.BlockSpec((GT, 56, 128), lambda i: (i, 0, 0)),
        pl.BlockSpec((T, 128, H), lambda i: (0, 0, 0)),
        pl.BlockSpec((1, H), lambda i: (0, 0)),
        pl.BlockSpec((H, O), lambda i: (0, 0)),
        pl.BlockSpec((1, O), lambda i: (0, 0)),
    ],
    out_specs=pl.BlockSpec((TB, O), lambda i: (i, 0)),
    out_shape=jax.ShapeDtypeStruct((B, O), jnp.float32),
)


def kernel(features, tables, W1, b1, W2, b2):
    v = features.astype(jnp.int32)
    offs = (jnp.arange(F, dtype=jnp.int32) * (4 * QP))[None, :]
    fi = offs + 4 * (v % QP) + v // QP
    fi28 = jnp.concatenate([fi, fi[:, 24:26]], axis=1)
    idx = fi28.reshape(NG, 8, T, 4).transpose(0, 3, 2, 1)
    idx = idx.reshape(NW, NCH, CH)
    idx = jnp.pad(idx, ((0, 0), (0, 0), (0, 128 - CH)))
    tT = jnp.transpose(tables, (0, 2, 1))
    tab128 = _tr(tT, tT, tT, tT)
    tab = tab128.reshape(F * 4 * QP, D)
    x3 = _sc_gather(tab, idx)
    w1p = jnp.concatenate(
        [W1, jnp.zeros((T * 128 - F * D, H), jnp.float32)]
    ).reshape(T, 128, H)
    return _mlp(x3, w1p, b1.reshape(1, H), W2, b2.reshape(1, O))
